# Initial kernel scaffold; baseline (speedup 1.0000x reference)
import jax
import jax.numpy as jnp
from jax.experimental import pallas as pl


def kernel(ploc, plabel, gloc, glabel, dboxes):
    raise NotImplementedError("write your pallas kernel here")


# speedup vs baseline: 2.4150x; 2.4150x over previous
import jax
import jax.numpy as jnp
from jax.experimental import pallas as pl
from jax.experimental.pallas import tpu as pltpu

_SCALE_XY = 10.0
_SCALE_WH = 5.0


def _smooth_l1(x):
    ax = jnp.abs(x)
    return jnp.where(ax < 1.0, 0.5 * x * x, ax - 0.5)


def _loss_body(ploc_ref, plabel_ref, gloc_ref, glabel_ref, dboxes_ref,
               out_ref, con_scr, mask_scr, pos_scr, loc_scr, conpos_scr,
               contot_scr, neg_scr):
    b = pl.program_id(0)
    nb = pl.num_programs(0)
    B, N = con_scr.shape

    x = plabel_ref[0]
    g = glabel_ref[0]
    C = x.shape[0]

    m = jnp.max(x, axis=0, keepdims=True)
    s = jnp.sum(jnp.exp(x - m), axis=0, keepdims=True)
    cls = jax.lax.broadcasted_iota(jnp.int32, (C, N), 0)
    gathered = jnp.sum(jnp.where(cls == g, x, 0.0), axis=0, keepdims=True)
    con = (m - gathered) + jnp.log(s)

    maskf = (g > 0).astype(jnp.float32)

    p = ploc_ref[0]
    gl = gloc_ref[0]
    d = dboxes_ref[0]
    vxy = _SCALE_XY * (gl[:2, :] - d[:2, :]) / d[2:, :]
    vwh = _SCALE_WH * jnp.log(gl[2:, :] / d[2:, :])
    sl1 = (jnp.sum(_smooth_l1(p[:2, :] - vxy), axis=0, keepdims=True)
           + jnp.sum(_smooth_l1(p[2:, :] - vwh), axis=0, keepdims=True))

    con_scr[pl.ds(b, 1), :] = con
    mask_scr[pl.ds(b, 1), :] = maskf
    pos_scr[pl.ds(b, 1), :] = jnp.sum(maskf, axis=1, keepdims=True)
    loc_scr[pl.ds(b, 1), :] = jnp.sum(maskf * sl1, axis=1, keepdims=True)
    conpos_scr[pl.ds(b, 1), :] = jnp.sum(maskf * con, axis=1, keepdims=True)
    contot_scr[pl.ds(b, 1), :] = jnp.sum(con, axis=1, keepdims=True)

    @pl.when(b == nb - 1)
    def _finalize():
        pos = pos_scr[...]
        contot = contot_scr[...]
        k = jnp.minimum(3.0 * pos, float(N))
        neg_scr[...] = contot
        needs = jnp.logical_and(k < float(N), k > 0.0)
        any_needs = jnp.sum(needs.astype(jnp.int32)) > 0

        @pl.when(any_needs)
        def _search():
            conf = con_scr[...]
            mf = mask_scr[...]
            con_neg = jnp.where(mf > 0.0, 0.0, conf)
            v = jax.lax.bitcast_convert_type(con_neg, jnp.int32)
            kint = k.astype(jnp.int32)

            lo0 = jnp.zeros((B, 1), jnp.int32)
            hi0 = jnp.full((B, 1), jnp.int32(0x7F7FFFFF))

            def vbody(_, lohi):
                lo, hi = lohi
                mid = lo + ((hi - lo + 1) >> 1)
                cnt = jnp.sum((v >= mid).astype(jnp.int32), axis=1,
                              keepdims=True)
                ge = cnt >= kint
                return jnp.where(ge, mid, lo), jnp.where(ge, hi, mid - 1)

            t, _ = jax.lax.fori_loop(0, 31, vbody, (lo0, hi0))

            gt = v > t
            cnt_gt = jnp.sum(gt.astype(jnp.int32), axis=1, keepdims=True)
            sum_gt = jnp.sum(jnp.where(gt, con_neg, 0.0), axis=1,
                             keepdims=True)
            chosen = kint - cnt_gt

            ties = v == t
            idx = jax.lax.broadcasted_iota(jnp.int32, (B, N), 1)
            lo1 = jnp.zeros((B, 1), jnp.int32)
            hi1 = jnp.full((B, 1), jnp.int32(N))

            def ibody(_, lohi):
                lo, hi = lohi
                mid = (lo + hi) >> 1
                cnt = jnp.sum(jnp.logical_and(ties, idx < mid)
                              .astype(jnp.int32), axis=1, keepdims=True)
                ge = cnt >= chosen
                return jnp.where(ge, lo, mid + 1), jnp.where(ge, mid, hi)

            mstop, _ = jax.lax.fori_loop(0, 14, ibody, (lo1, hi1))
            tie_take = jnp.logical_and(ties, idx < mstop)
            tie_sum = jnp.sum(jnp.where(tie_take, conf, 0.0), axis=1,
                              keepdims=True)
            neg_scr[...] = jnp.where(needs, sum_gt + tie_sum, contot)

        total = loc_scr[...] + conpos_scr[...] + neg_scr[...]
        num_mask = (pos > 0.0).astype(jnp.float32)
        per_row = total * num_mask / jnp.maximum(pos, 1e-6)
        out_ref[...] = jnp.sum(per_row, axis=0, keepdims=True) / float(B)


def kernel(ploc, plabel, gloc, glabel, dboxes):
    B, C, N = plabel.shape
    glabel3 = glabel.reshape(B, 1, N).astype(jnp.int32)
    out = pl.pallas_call(
        _loss_body,
        grid=(B,),
        in_specs=[
            pl.BlockSpec((1, 4, N), lambda b: (b, 0, 0)),
            pl.BlockSpec((1, C, N), lambda b: (b, 0, 0)),
            pl.BlockSpec((1, 4, N), lambda b: (b, 0, 0)),
            pl.BlockSpec((1, 1, N), lambda b: (b, 0, 0)),
            pl.BlockSpec((1, 4, N), lambda b: (0, 0, 0)),
        ],
        out_specs=pl.BlockSpec((1, 1), lambda b: (0, 0)),
        out_shape=jax.ShapeDtypeStruct((1, 1), jnp.float32),
        scratch_shapes=[
            pltpu.VMEM((B, N), jnp.float32),
            pltpu.VMEM((B, N), jnp.float32),
            pltpu.VMEM((B, 1), jnp.float32),
            pltpu.VMEM((B, 1), jnp.float32),
            pltpu.VMEM((B, 1), jnp.float32),
            pltpu.VMEM((B, 1), jnp.float32),
            pltpu.VMEM((B, 1), jnp.float32),
        ],
    )(ploc, plabel, gloc, glabel3, dboxes)
    return out.reshape(())


# speedup vs baseline: 7.0536x; 2.9208x over previous
import jax
import jax.numpy as jnp
from jax.experimental import pallas as pl
from jax.experimental.pallas import tpu as pltpu

_SCALE_XY = 10.0
_SCALE_WH = 5.0


def _smooth_l1(x):
    ax = jnp.abs(x)
    return jnp.where(ax < 1.0, 0.5 * x * x, ax - 0.5)


def _make_body(N, NT):
    def body(plabel_ref, glabel_ref, ploc_ref, gloc_ref, dbc_ref, out_ref,
             con_scr, mask_scr, pos_acc, loc_acc, conpos_acc, contot_acc,
             neg_scr):
        i = pl.program_id(0)
        ni = pl.num_programs(0)
        NP, B = con_scr.shape

        x = plabel_ref[...]
        g = glabel_ref[...]
        C = x.shape[0]
        nt = g.shape[0]

        rid = i * NT + jax.lax.broadcasted_iota(jnp.int32, (nt, 1), 0)
        valid = rid < N

        s = jnp.sum(jnp.exp(x), axis=0)
        cls = jax.lax.broadcasted_iota(jnp.int32, (C, nt, B), 0)
        gathered = jnp.sum(jnp.where(cls == g[None], x, 0.0), axis=0)
        con = jnp.maximum(jnp.log(s) - gathered, 0.0)
        con = jnp.where(valid, con, 0.0)
        maskf = jnp.where(valid, (g > 0).astype(jnp.float32), 0.0)

        p = ploc_ref[...]
        gl = gloc_ref[...]
        d = dbc_ref[...]
        vxy = _SCALE_XY * (gl[:, :2, :] - d[:, :2, :]) / d[:, 2:, :]
        vwh = _SCALE_WH * jnp.log(gl[:, 2:, :] / d[:, 2:, :])
        sl1 = (jnp.sum(_smooth_l1(p[:, :2, :] - vxy), axis=1)
               + jnp.sum(_smooth_l1(p[:, 2:, :] - vwh), axis=1))
        loc_part = jnp.where(valid, maskf * sl1, 0.0)

        con_scr[pl.ds(i * NT, NT), :] = con
        mask_scr[pl.ds(i * NT, NT), :] = maskf

        @pl.when(i == 0)
        def _init():
            pos_acc[...] = jnp.zeros_like(pos_acc)
            loc_acc[...] = jnp.zeros_like(loc_acc)
            conpos_acc[...] = jnp.zeros_like(conpos_acc)
            contot_acc[...] = jnp.zeros_like(contot_acc)

        pos_acc[...] += jnp.sum(maskf, axis=0, keepdims=True)
        loc_acc[...] += jnp.sum(loc_part, axis=0, keepdims=True)
        conpos_acc[...] += jnp.sum(maskf * con, axis=0, keepdims=True)
        contot_acc[...] += jnp.sum(con, axis=0, keepdims=True)

        @pl.when(i == ni - 1)
        def _finalize():
            pos = pos_acc[...]
            contot = contot_acc[...]
            k = jnp.minimum(3.0 * pos, float(N))
            neg_scr[...] = contot
            needs = jnp.logical_and(k < float(N), k > 0.0)
            any_needs = jnp.sum(needs.astype(jnp.int32)) > 0

            @pl.when(any_needs)
            def _search():
                conf = con_scr[...]
                mf = mask_scr[...]
                con_neg = jnp.where(mf > 0.0, 0.0, conf)
                v = jax.lax.bitcast_convert_type(con_neg, jnp.int32)
                kint = k.astype(jnp.int32)
                idxn = jax.lax.broadcasted_iota(jnp.int32, (NP, B), 0)

                lo0 = jnp.zeros((1, B), jnp.int32)
                hi0 = jnp.full((1, B), jnp.int32(0x7F7FFFFF))

                def vbody(_, lohi):
                    lo, hi = lohi
                    mid = lo + ((hi - lo + 1) >> 1)
                    cnt = jnp.sum((v >= mid).astype(jnp.int32), axis=0,
                                  keepdims=True)
                    ge = cnt >= kint
                    return jnp.where(ge, mid, lo), jnp.where(ge, hi, mid - 1)

                t, _ = jax.lax.fori_loop(0, 31, vbody, (lo0, hi0))

                gt = v > t
                cnt_gt = jnp.sum(gt.astype(jnp.int32), axis=0, keepdims=True)
                sum_gt = jnp.sum(jnp.where(gt, con_neg, 0.0), axis=0,
                                 keepdims=True)
                chosen = kint - cnt_gt

                ties = jnp.logical_and(v == t, idxn < N)
                lo1 = jnp.zeros((1, B), jnp.int32)
                hi1 = jnp.full((1, B), jnp.int32(NP))

                def ibody(_, lohi):
                    lo, hi = lohi
                    mid = (lo + hi) >> 1
                    cnt = jnp.sum(jnp.logical_and(ties, idxn < mid)
                                  .astype(jnp.int32), axis=0, keepdims=True)
                    ge = cnt >= chosen
                    return jnp.where(ge, lo, mid + 1), jnp.where(ge, mid, hi)

                mstop, _ = jax.lax.fori_loop(0, 14, ibody, (lo1, hi1))
                tie_take = jnp.logical_and(ties, idxn < mstop)
                tie_sum = jnp.sum(jnp.where(tie_take, conf, 0.0), axis=0,
                                  keepdims=True)
                neg_scr[...] = jnp.where(needs, sum_gt + tie_sum, contot)

            total = loc_acc[...] + conpos_acc[...] + neg_scr[...]
            num_mask = (pos > 0.0).astype(jnp.float32)
            per_row = total * num_mask / jnp.maximum(pos, 1e-6)
            out_ref[...] = jnp.sum(per_row, axis=1, keepdims=True) / float(B)

    return body


def kernel(ploc, plabel, gloc, glabel, dboxes):
    B, C, N = plabel.shape
    NT = 64
    NB = (N + NT - 1) // NT
    NP = NB * NT

    plabel_t = jnp.transpose(plabel, (1, 2, 0))
    glabel_t = jnp.transpose(glabel, (1, 0))
    ploc_t = jnp.transpose(ploc, (2, 1, 0))
    gloc_t = jnp.transpose(gloc, (2, 1, 0))
    dbc = jnp.broadcast_to(jnp.transpose(dboxes, (2, 1, 0)), (N, 4, B))

    out = pl.pallas_call(
        _make_body(N, NT),
        grid=(NB,),
        in_specs=[
            pl.BlockSpec((C, NT, B), lambda i: (0, i, 0)),
            pl.BlockSpec((NT, B), lambda i: (i, 0)),
            pl.BlockSpec((NT, 4, B), lambda i: (i, 0, 0)),
            pl.BlockSpec((NT, 4, B), lambda i: (i, 0, 0)),
            pl.BlockSpec((NT, 4, B), lambda i: (i, 0, 0)),
        ],
        out_specs=pl.BlockSpec((1, 1), lambda i: (0, 0)),
        out_shape=jax.ShapeDtypeStruct((1, 1), jnp.float32),
        scratch_shapes=[
            pltpu.VMEM((NP, B), jnp.float32),
            pltpu.VMEM((NP, B), jnp.float32),
            pltpu.VMEM((1, B), jnp.float32),
            pltpu.VMEM((1, B), jnp.float32),
            pltpu.VMEM((1, B), jnp.float32),
            pltpu.VMEM((1, B), jnp.float32),
            pltpu.VMEM((1, B), jnp.float32),
        ],
    )(plabel_t, glabel_t, ploc_t, gloc_t, dbc)
    return out.reshape(())


# speedup vs baseline: 8.4788x; 1.2020x over previous
import jax
import jax.numpy as jnp
from jax.experimental import pallas as pl
from jax.experimental.pallas import tpu as pltpu

_SCALE_XY = 10.0
_SCALE_WH = 5.0


def _smooth_l1(x):
    ax = jnp.abs(x)
    return jnp.where(ax < 1.0, 0.5 * x * x, ax - 0.5)


def _make_body(N, NT):
    def body(plabel_ref, glabel_ref, ploc_ref, gloc_ref, dbc_ref, out_ref,
             con_scr, mask_scr, pos_acc, loc_acc, conpos_acc, contot_acc,
             neg_scr):
        i = pl.program_id(0)
        ni = pl.num_programs(0)
        NP, B = con_scr.shape

        x = plabel_ref[...]
        g = glabel_ref[...]
        C = x.shape[0]
        nt = g.shape[0]

        rid = i * NT + jax.lax.broadcasted_iota(jnp.int32, (nt, 1), 0)
        valid = rid < N

        s = jnp.sum(jnp.exp(x), axis=0)
        cls = jax.lax.broadcasted_iota(jnp.int32, (C, nt, B), 0)
        gathered = jnp.sum(jnp.where(cls == g[None], x, 0.0), axis=0)
        con = jnp.maximum(jnp.log(s) - gathered, 0.0)
        con = jnp.where(valid, con, 0.0)
        maskf = jnp.where(valid, (g > 0).astype(jnp.float32), 0.0)

        p = ploc_ref[...]
        gl = gloc_ref[...]
        d = dbc_ref[...]
        vxy = _SCALE_XY * (gl[:, :2, :] - d[:, :2, :]) / d[:, 2:, :]
        vwh = _SCALE_WH * jnp.log(gl[:, 2:, :] / d[:, 2:, :])
        sl1 = (jnp.sum(_smooth_l1(p[:, :2, :] - vxy), axis=1)
               + jnp.sum(_smooth_l1(p[:, 2:, :] - vwh), axis=1))
        loc_part = jnp.where(valid, maskf * sl1, 0.0)

        con_scr[pl.ds(i * NT, NT), :] = con
        mask_scr[pl.ds(i * NT, NT), :] = maskf

        @pl.when(i == 0)
        def _init():
            pos_acc[...] = jnp.zeros_like(pos_acc)
            loc_acc[...] = jnp.zeros_like(loc_acc)
            conpos_acc[...] = jnp.zeros_like(conpos_acc)
            contot_acc[...] = jnp.zeros_like(contot_acc)

        pos_acc[...] += jnp.sum(maskf, axis=0, keepdims=True)
        loc_acc[...] += jnp.sum(loc_part, axis=0, keepdims=True)
        conpos_acc[...] += jnp.sum(maskf * con, axis=0, keepdims=True)
        contot_acc[...] += jnp.sum(con, axis=0, keepdims=True)

        @pl.when(i == ni - 1)
        def _finalize():
            pos = pos_acc[...]
            contot = contot_acc[...]
            k = jnp.minimum(3.0 * pos, float(N))
            neg_scr[...] = contot
            needs = jnp.logical_and(k < float(N), k > 0.0)
            any_needs = jnp.sum(needs.astype(jnp.int32)) > 0

            @pl.when(any_needs)
            def _search():
                conf = con_scr[...]
                mf = mask_scr[...]
                con_neg = jnp.where(mf > 0.0, 0.0, conf)
                v = jax.lax.bitcast_convert_type(con_neg, jnp.int32)
                kint = k.astype(jnp.int32)
                idxn = jax.lax.broadcasted_iota(jnp.int32, (NP, B), 0)

                lo0 = jnp.zeros((1, B), jnp.int32)
                hi0 = jnp.full((1, B), jnp.int32(0x7F7FFFFF))

                def vbody(_, lohi):
                    lo, hi = lohi
                    mid = lo + ((hi - lo + 1) >> 1)
                    cnt = jnp.sum((v >= mid).astype(jnp.int32), axis=0,
                                  keepdims=True)
                    ge = cnt >= kint
                    return jnp.where(ge, mid, lo), jnp.where(ge, hi, mid - 1)

                t, _ = jax.lax.fori_loop(0, 31, vbody, (lo0, hi0))

                gt = v > t
                cnt_gt = jnp.sum(gt.astype(jnp.int32), axis=0, keepdims=True)
                sum_gt = jnp.sum(jnp.where(gt, con_neg, 0.0), axis=0,
                                 keepdims=True)
                chosen = kint - cnt_gt

                ties = jnp.logical_and(v == t, idxn < N)
                lo1 = jnp.zeros((1, B), jnp.int32)
                hi1 = jnp.full((1, B), jnp.int32(NP))

                def ibody(_, lohi):
                    lo, hi = lohi
                    mid = (lo + hi) >> 1
                    cnt = jnp.sum(jnp.logical_and(ties, idxn < mid)
                                  .astype(jnp.int32), axis=0, keepdims=True)
                    ge = cnt >= chosen
                    return jnp.where(ge, lo, mid + 1), jnp.where(ge, mid, hi)

                mstop, _ = jax.lax.fori_loop(0, 14, ibody, (lo1, hi1))
                tie_take = jnp.logical_and(ties, idxn < mstop)
                tie_sum = jnp.sum(jnp.where(tie_take, conf, 0.0), axis=0,
                                  keepdims=True)
                neg_scr[...] = jnp.where(needs, sum_gt + tie_sum, contot)

            total = loc_acc[...] + conpos_acc[...] + neg_scr[...]
            num_mask = (pos > 0.0).astype(jnp.float32)
            per_row = total * num_mask / jnp.maximum(pos, 1e-6)
            out_ref[...] = jnp.sum(per_row, axis=1, keepdims=True) / float(B)

    return body


def kernel(ploc, plabel, gloc, glabel, dboxes):
    B, C, N = plabel.shape
    NT = 128
    NB = (N + NT - 1) // NT
    NP = NB * NT

    plabel_t = jnp.transpose(plabel, (1, 2, 0))
    glabel_t = jnp.transpose(glabel, (1, 0))
    ploc_t = jnp.transpose(ploc, (2, 1, 0))
    gloc_t = jnp.transpose(gloc, (2, 1, 0))
    dbc = jnp.broadcast_to(jnp.transpose(dboxes, (2, 1, 0)), (N, 4, B))

    out = pl.pallas_call(
        _make_body(N, NT),
        grid=(NB,),
        in_specs=[
            pl.BlockSpec((C, NT, B), lambda i: (0, i, 0)),
            pl.BlockSpec((NT, B), lambda i: (i, 0)),
            pl.BlockSpec((NT, 4, B), lambda i: (i, 0, 0)),
            pl.BlockSpec((NT, 4, B), lambda i: (i, 0, 0)),
            pl.BlockSpec((NT, 4, B), lambda i: (i, 0, 0)),
        ],
        out_specs=pl.BlockSpec((1, 1), lambda i: (0, 0)),
        out_shape=jax.ShapeDtypeStruct((1, 1), jnp.float32),
        scratch_shapes=[
            pltpu.VMEM((NP, B), jnp.float32),
            pltpu.VMEM((NP, B), jnp.float32),
            pltpu.VMEM((1, B), jnp.float32),
            pltpu.VMEM((1, B), jnp.float32),
            pltpu.VMEM((1, B), jnp.float32),
            pltpu.VMEM((1, B), jnp.float32),
            pltpu.VMEM((1, B), jnp.float32),
        ],
    )(plabel_t, glabel_t, ploc_t, gloc_t, dbc)
    return out.reshape(())


# speedup vs baseline: 9.3067x; 1.0976x over previous
import jax
import jax.numpy as jnp
from jax.experimental import pallas as pl
from jax.experimental.pallas import tpu as pltpu

_SCALE_XY = 10.0
_SCALE_WH = 5.0


def _smooth_l1(x):
    ax = jnp.abs(x)
    return jnp.where(ax < 1.0, 0.5 * x * x, ax - 0.5)


def _make_body(N, NT):
    def body(plabel_ref, glabel_ref, ploc_ref, gloc_ref, dbc_ref, out_ref,
             con_scr, mask_scr, pos_acc, loc_acc, conpos_acc, contot_acc,
             neg_scr):
        i = pl.program_id(0)
        ni = pl.num_programs(0)
        NP, B = con_scr.shape

        x = plabel_ref[...]
        g = glabel_ref[...]
        C = x.shape[0]
        nt = g.shape[0]

        rid = i * NT + jax.lax.broadcasted_iota(jnp.int32, (nt, 1), 0)
        valid = rid < N

        s = jnp.sum(jnp.exp(x), axis=0)
        cls = jax.lax.broadcasted_iota(jnp.int32, (C, nt, B), 0)
        gathered = jnp.sum(jnp.where(cls == g[None], x, 0.0), axis=0)
        con = jnp.maximum(jnp.log(s) - gathered, 0.0)
        con = jnp.where(valid, con, 0.0)
        maskf = jnp.where(valid, (g > 0).astype(jnp.float32), 0.0)

        p = ploc_ref[...]
        gl = gloc_ref[...]
        d = dbc_ref[...]
        vxy = _SCALE_XY * (gl[:, :2, :] - d[:, :2, :]) / d[:, 2:, :]
        vwh = _SCALE_WH * jnp.log(gl[:, 2:, :] / d[:, 2:, :])
        sl1 = (jnp.sum(_smooth_l1(p[:, :2, :] - vxy), axis=1)
               + jnp.sum(_smooth_l1(p[:, 2:, :] - vwh), axis=1))
        loc_part = jnp.where(valid, maskf * sl1, 0.0)

        con_scr[pl.ds(i * NT, NT), :] = con
        mask_scr[pl.ds(i * NT, NT), :] = maskf

        @pl.when(i == 0)
        def _init():
            pos_acc[...] = jnp.zeros_like(pos_acc)
            loc_acc[...] = jnp.zeros_like(loc_acc)
            conpos_acc[...] = jnp.zeros_like(conpos_acc)
            contot_acc[...] = jnp.zeros_like(contot_acc)

        pos_acc[...] += jnp.sum(maskf, axis=0, keepdims=True)
        loc_acc[...] += jnp.sum(loc_part, axis=0, keepdims=True)
        conpos_acc[...] += jnp.sum(maskf * con, axis=0, keepdims=True)
        contot_acc[...] += jnp.sum(con, axis=0, keepdims=True)

        @pl.when(i == ni - 1)
        def _finalize():
            pos = pos_acc[...]
            contot = contot_acc[...]
            k = jnp.minimum(3.0 * pos, float(N))
            neg_scr[...] = contot
            needs = jnp.logical_and(k < float(N), k > 0.0)
            any_needs = jnp.sum(needs.astype(jnp.int32)) > 0

            @pl.when(any_needs)
            def _search():
                conf = con_scr[...]
                mf = mask_scr[...]
                con_neg = jnp.where(mf > 0.0, 0.0, conf)
                v = jax.lax.bitcast_convert_type(con_neg, jnp.int32)
                kint = k.astype(jnp.int32)
                idxn = jax.lax.broadcasted_iota(jnp.int32, (NP, B), 0)

                lo0 = jnp.zeros((1, B), jnp.int32)
                hi0 = jnp.full((1, B), jnp.int32(0x7F7FFFFF))

                def vbody(_, lohi):
                    lo, hi = lohi
                    mid = lo + ((hi - lo + 1) >> 1)
                    cnt = jnp.sum((v >= mid).astype(jnp.int32), axis=0,
                                  keepdims=True)
                    ge = cnt >= kint
                    return jnp.where(ge, mid, lo), jnp.where(ge, hi, mid - 1)

                t, _ = jax.lax.fori_loop(0, 31, vbody, (lo0, hi0))

                gt = v > t
                cnt_gt = jnp.sum(gt.astype(jnp.int32), axis=0, keepdims=True)
                sum_gt = jnp.sum(jnp.where(gt, con_neg, 0.0), axis=0,
                                 keepdims=True)
                chosen = kint - cnt_gt

                ties = jnp.logical_and(v == t, idxn < N)
                lo1 = jnp.zeros((1, B), jnp.int32)
                hi1 = jnp.full((1, B), jnp.int32(NP))

                def ibody(_, lohi):
                    lo, hi = lohi
                    mid = (lo + hi) >> 1
                    cnt = jnp.sum(jnp.logical_and(ties, idxn < mid)
                                  .astype(jnp.int32), axis=0, keepdims=True)
                    ge = cnt >= chosen
                    return jnp.where(ge, lo, mid + 1), jnp.where(ge, mid, hi)

                mstop, _ = jax.lax.fori_loop(0, 14, ibody, (lo1, hi1))
                tie_take = jnp.logical_and(ties, idxn < mstop)
                tie_sum = jnp.sum(jnp.where(tie_take, conf, 0.0), axis=0,
                                  keepdims=True)
                neg_scr[...] = jnp.where(needs, sum_gt + tie_sum, contot)

            total = loc_acc[...] + conpos_acc[...] + neg_scr[...]
            num_mask = (pos > 0.0).astype(jnp.float32)
            per_row = total * num_mask / jnp.maximum(pos, 1e-6)
            out_ref[...] = jnp.sum(per_row, axis=1, keepdims=True) / float(B)

    return body


def kernel(ploc, plabel, gloc, glabel, dboxes):
    B, C, N = plabel.shape
    NT = 256
    NB = (N + NT - 1) // NT
    NP = NB * NT

    plabel_t = jnp.transpose(plabel, (1, 2, 0))
    glabel_t = jnp.transpose(glabel, (1, 0))
    ploc_t = jnp.transpose(ploc, (2, 1, 0))
    gloc_t = jnp.transpose(gloc, (2, 1, 0))
    dbc = jnp.broadcast_to(jnp.transpose(dboxes, (2, 1, 0)), (N, 4, B))

    out = pl.pallas_call(
        _make_body(N, NT),
        grid=(NB,),
        in_specs=[
            pl.BlockSpec((C, NT, B), lambda i: (0, i, 0)),
            pl.BlockSpec((NT, B), lambda i: (i, 0)),
            pl.BlockSpec((NT, 4, B), lambda i: (i, 0, 0)),
            pl.BlockSpec((NT, 4, B), lambda i: (i, 0, 0)),
            pl.BlockSpec((NT, 4, B), lambda i: (i, 0, 0)),
        ],
        out_specs=pl.BlockSpec((1, 1), lambda i: (0, 0)),
        out_shape=jax.ShapeDtypeStruct((1, 1), jnp.float32),
        scratch_shapes=[
            pltpu.VMEM((NP, B), jnp.float32),
            pltpu.VMEM((NP, B), jnp.float32),
            pltpu.VMEM((1, B), jnp.float32),
            pltpu.VMEM((1, B), jnp.float32),
            pltpu.VMEM((1, B), jnp.float32),
            pltpu.VMEM((1, B), jnp.float32),
            pltpu.VMEM((1, B), jnp.float32),
        ],
    )(plabel_t, glabel_t, ploc_t, gloc_t, dbc)
    return out.reshape(())


# speedup vs baseline: 9.6916x; 1.0414x over previous
import jax
import jax.numpy as jnp
from jax.experimental import pallas as pl
from jax.experimental.pallas import tpu as pltpu

_SCALE_XY = 10.0
_SCALE_WH = 5.0


def _smooth_l1(x):
    ax = jnp.abs(x)
    return jnp.where(ax < 1.0, 0.5 * x * x, ax - 0.5)


def _make_body(N, NT):
    def body(plabel_ref, glabel_ref, ploc_ref, gloc_ref, dbc_ref, out_ref,
             con_scr, mask_scr, pos_acc, loc_acc, conpos_acc, contot_acc,
             neg_scr):
        i = pl.program_id(0)
        ni = pl.num_programs(0)
        NP, B = con_scr.shape

        x = plabel_ref[...]
        g = glabel_ref[...]
        C = x.shape[0]
        nt = g.shape[0]

        rid = i * NT + jax.lax.broadcasted_iota(jnp.int32, (nt, 1), 0)
        valid = rid < N

        s = jnp.sum(jnp.exp(x), axis=0)
        cls = jax.lax.broadcasted_iota(jnp.int32, (C, nt, B), 0)
        gathered = jnp.sum(jnp.where(cls == g[None], x, 0.0), axis=0)
        con = jnp.maximum(jnp.log(s) - gathered, 0.0)
        con = jnp.where(valid, con, 0.0)
        maskf = jnp.where(valid, (g > 0).astype(jnp.float32), 0.0)

        p = ploc_ref[...]
        gl = gloc_ref[...]
        d = dbc_ref[...]
        vxy = _SCALE_XY * (gl[:, :2, :] - d[:, :2, :]) / d[:, 2:, :]
        vwh = _SCALE_WH * jnp.log(gl[:, 2:, :] / d[:, 2:, :])
        sl1 = (jnp.sum(_smooth_l1(p[:, :2, :] - vxy), axis=1)
               + jnp.sum(_smooth_l1(p[:, 2:, :] - vwh), axis=1))
        loc_part = jnp.where(valid, maskf * sl1, 0.0)

        con_scr[pl.ds(i * NT, NT), :] = con
        mask_scr[pl.ds(i * NT, NT), :] = maskf

        @pl.when(i == 0)
        def _init():
            pos_acc[...] = jnp.zeros_like(pos_acc)
            loc_acc[...] = jnp.zeros_like(loc_acc)
            conpos_acc[...] = jnp.zeros_like(conpos_acc)
            contot_acc[...] = jnp.zeros_like(contot_acc)

        pos_acc[...] += jnp.sum(maskf, axis=0, keepdims=True)
        loc_acc[...] += jnp.sum(loc_part, axis=0, keepdims=True)
        conpos_acc[...] += jnp.sum(maskf * con, axis=0, keepdims=True)
        contot_acc[...] += jnp.sum(con, axis=0, keepdims=True)

        @pl.when(i == ni - 1)
        def _finalize():
            pos = pos_acc[...]
            contot = contot_acc[...]
            k = jnp.minimum(3.0 * pos, float(N))
            neg_scr[...] = contot
            needs = jnp.logical_and(k < float(N), k > 0.0)
            any_needs = jnp.sum(needs.astype(jnp.int32)) > 0

            @pl.when(any_needs)
            def _search():
                conf = con_scr[...]
                mf = mask_scr[...]
                con_neg = jnp.where(mf > 0.0, 0.0, conf)
                v = jax.lax.bitcast_convert_type(con_neg, jnp.int32)
                kint = k.astype(jnp.int32)
                idxn = jax.lax.broadcasted_iota(jnp.int32, (NP, B), 0)

                lo0 = jnp.zeros((1, B), jnp.int32)
                hi0 = jnp.full((1, B), jnp.int32(0x7F7FFFFF))

                def vbody(_, lohi):
                    lo, hi = lohi
                    mid = lo + ((hi - lo + 1) >> 1)
                    cnt = jnp.sum((v >= mid).astype(jnp.int32), axis=0,
                                  keepdims=True)
                    ge = cnt >= kint
                    return jnp.where(ge, mid, lo), jnp.where(ge, hi, mid - 1)

                t, _ = jax.lax.fori_loop(0, 31, vbody, (lo0, hi0))

                gt = v > t
                cnt_gt = jnp.sum(gt.astype(jnp.int32), axis=0, keepdims=True)
                sum_gt = jnp.sum(jnp.where(gt, con_neg, 0.0), axis=0,
                                 keepdims=True)
                chosen = kint - cnt_gt

                ties = jnp.logical_and(v == t, idxn < N)
                lo1 = jnp.zeros((1, B), jnp.int32)
                hi1 = jnp.full((1, B), jnp.int32(NP))

                def ibody(_, lohi):
                    lo, hi = lohi
                    mid = (lo + hi) >> 1
                    cnt = jnp.sum(jnp.logical_and(ties, idxn < mid)
                                  .astype(jnp.int32), axis=0, keepdims=True)
                    ge = cnt >= chosen
                    return jnp.where(ge, lo, mid + 1), jnp.where(ge, mid, hi)

                mstop, _ = jax.lax.fori_loop(0, 14, ibody, (lo1, hi1))
                tie_take = jnp.logical_and(ties, idxn < mstop)
                tie_sum = jnp.sum(jnp.where(tie_take, conf, 0.0), axis=0,
                                  keepdims=True)
                neg_scr[...] = jnp.where(needs, sum_gt + tie_sum, contot)

            total = loc_acc[...] + conpos_acc[...] + neg_scr[...]
            num_mask = (pos > 0.0).astype(jnp.float32)
            per_row = total * num_mask / jnp.maximum(pos, 1e-6)
            out_ref[...] = jnp.sum(per_row, axis=1, keepdims=True) / float(B)

    return body


def kernel(ploc, plabel, gloc, glabel, dboxes):
    B, C, N = plabel.shape
    NT = 384
    NB = (N + NT - 1) // NT
    NP = NB * NT

    plabel_t = jnp.transpose(plabel, (1, 2, 0))
    glabel_t = jnp.transpose(glabel, (1, 0))
    ploc_t = jnp.transpose(ploc, (2, 1, 0))
    gloc_t = jnp.transpose(gloc, (2, 1, 0))
    dbc = jnp.broadcast_to(jnp.transpose(dboxes, (2, 1, 0)), (N, 4, B))

    out = pl.pallas_call(
        _make_body(N, NT),
        grid=(NB,),
        in_specs=[
            pl.BlockSpec((C, NT, B), lambda i: (0, i, 0)),
            pl.BlockSpec((NT, B), lambda i: (i, 0)),
            pl.BlockSpec((NT, 4, B), lambda i: (i, 0, 0)),
            pl.BlockSpec((NT, 4, B), lambda i: (i, 0, 0)),
            pl.BlockSpec((NT, 4, B), lambda i: (i, 0, 0)),
        ],
        out_specs=pl.BlockSpec((1, 1), lambda i: (0, 0)),
        out_shape=jax.ShapeDtypeStruct((1, 1), jnp.float32),
        scratch_shapes=[
            pltpu.VMEM((NP, B), jnp.float32),
            pltpu.VMEM((NP, B), jnp.float32),
            pltpu.VMEM((1, B), jnp.float32),
            pltpu.VMEM((1, B), jnp.float32),
            pltpu.VMEM((1, B), jnp.float32),
            pltpu.VMEM((1, B), jnp.float32),
            pltpu.VMEM((1, B), jnp.float32),
        ],
    )(plabel_t, glabel_t, ploc_t, gloc_t, dbc)
    return out.reshape(())
